# baseline (speedup 1.0000x reference)
import functools

import jax
import jax.numpy as jnp
from jax import lax
from jax.experimental import pallas as pl
from jax.experimental.pallas import tpu as pltpu
from jax.experimental.pallas import tpu_sc as plsc

B = 128
N = 35709
K = 64
C = 17
F = 51
NLM = C + F
NCAND = 1152
VB = 2432
NBF = 14
TAILV = NBF * VB
TAILW = N - TAILV
NB0 = 7
NB1 = 7
OUTW = NLM * 3
EXTW = 24


def _extract_block(buf, ext, vs_v, bs_v, j, d, iota, lane_hi, b_i):
    win = bs_v[pl.ds(j, 16)]
    s_lo = win[0]
    s_hi = win[1]
    vt = j * VB
    npairs = (s_hi - s_lo + 1) // 2

    def pair(i, carry):
        cand = s_lo + 2 * i + lane_hi
        msk = cand < s_hi
        v = plsc.load_gather(vs_v, [cand], mask=msk)
        col = v - vt
        val = plsc.load_gather(buf, [b_i, col], mask=msk)
        dst = cand * EXTW + d * 8 + b_i
        plsc.store_scatter(ext, [dst], val, mask=msk)
        return carry
    lax.fori_loop(0, npairs, pair, 0)


def _sc_body(tab_hbm, vs_hbm, sp_hbm, bs_hbm, out_hbm,
             vs_v, sp_v, bs_v, bufa, bufb, tbuf, ext, ext2, srcrow, outacc,
             shared, sema, semb):
    c = lax.axis_index("c")
    s = lax.axis_index("s")
    bbl = s % 8
    h = s // 8
    gbb = c * 8 + bbl
    b0 = gbb * 8
    iota = lax.iota(jnp.int32, 16)
    lane_hi = (iota >= 8).astype(jnp.int32)
    b_i = iota % 8

    pltpu.sync_copy(vs_hbm, vs_v.at[pl.ds(0, NCAND)])
    pltpu.sync_copy(sp_hbm, sp_v)
    pltpu.sync_copy(bs_hbm, bs_v)

    @pl.when(h == 1)
    def _():
        for d in range(3):
            row = pl.multiple_of(d * 128 + b0, 8)
            pltpu.sync_copy(
                tab_hbm.at[pl.ds(row, 8), pl.ds(TAILV, TAILW)], tbuf)
            _extract_block(tbuf, ext, vs_v, bs_v,
                           jnp.int32(NBF), d, iota, lane_hi, b_i)

    jb = jnp.where(h == 0, 0, NB0)
    nb = jnp.where(h == 0, NB0, NB1)
    total = 3 * nb

    def blk_slice(t):
        d = t // nb
        j = jb + t % nb
        row = pl.multiple_of(d * 128 + b0, 8)
        vt = pl.multiple_of((jb + t % nb) * VB, 128)
        return tab_hbm.at[pl.ds(row, 8), pl.ds(vt, VB)], d, j

    def issue(t, buf, sem):
        @pl.when(t < total)
        def _():
            src, _, _ = blk_slice(t)
            pltpu.async_copy(src, buf, sem)

    def drain_extract(t, buf, sem):
        @pl.when(t < total)
        def _():
            src, d, j = blk_slice(t)
            pltpu.make_async_copy(src, buf, sem).wait()
            _extract_block(buf, ext, vs_v, bs_v, j, d, iota, lane_hi, b_i)

    issue(jnp.int32(0), bufa, sema)

    def pipe(i, carry):
        t0 = 2 * i
        t1 = t0 + 1
        issue(t1, bufb, semb)
        drain_extract(t0, bufa, sema)
        issue(t0 + 2, bufa, sema)
        drain_extract(t1, bufb, semb)
        return carry
    lax.fori_loop(0, 11, pipe, 0)

    @pl.when(h == 1)
    def _():
        pltpu.sync_copy(ext, shared.at[bbl])
    plsc.subcore_barrier()

    @pl.when(h == 0)
    def _():
        pltpu.sync_copy(shared.at[bbl], ext2)
        n0 = bs_v[pl.ds(NB0, 16)][0]

        for ss in range(5):
            srcrow[pl.ds(ss * 16, 16)] = (ss * 16 + (K * C - C)) + iota

        def merged_gather(addr, sp):
            v0 = plsc.load_gather(ext, [addr])
            v1 = plsc.load_gather(ext2, [addr])
            return jnp.where(sp < n0, v0, v1)

        def batch_body(bi, carry):
            def col_body(cc, c2):
                comp = jnp.where(cc == 8, 1, 0)
                sgn = jnp.where(cc < 8, 1.0, -1.0)
                base = cc * K

                def chunk(q, st):
                    bk, br = st
                    slots = base + q * 16 + iota
                    sp = plsc.load_gather(sp_v, [slots])
                    val = merged_gather(sp * EXTW + comp * 8 + bi, sp)
                    key = val * sgn
                    upd = key < bk
                    return (jnp.where(upd, key, bk),
                            jnp.where(upd, slots, br))

                best_key, best_row = lax.fori_loop(
                    0, K // 16, chunk,
                    (jnp.full((16,), jnp.inf, jnp.float32),
                     jnp.zeros((16,), jnp.int32)))
                m = jnp.min(best_key)
                win = jnp.min(jnp.where(best_key == m, best_row,
                                        jnp.int32(2 ** 30)))
                plsc.store_scatter(srcrow, [jnp.full((16,), cc, jnp.int32)],
                                   jnp.full((16,), win, jnp.int32),
                                   mask=iota == 0)
                return c2
            lax.fori_loop(0, C, col_body, 0)

            def slot_body(t, c2):
                p = t * 16 + iota
                i = p // 3
                dc = p - i * 3
                slot = plsc.load_gather(srcrow, [i])
                sp = plsc.load_gather(sp_v, [slot])
                val = merged_gather(sp * EXTW + dc * 8 + bi, sp)
                plsc.store_scatter(outacc, [bi * OUTW + p], val)
                return c2
            lax.fori_loop(0, 13, slot_body, 0)
            return carry
        lax.fori_loop(0, 8, batch_body, 0)

        pltpu.sync_copy(outacc.at[pl.ds(0, 8 * OUTW)],
                        out_hbm.at[pl.ds(gbb * 8 * OUTW, 8 * OUTW)])


def kernel(batch_cam_vps, contour_idx, inface_idx):
    tab = jnp.transpose(batch_cam_vps, (2, 0, 1)).reshape(3 * B, N)

    ci = contour_idx.astype(jnp.int32)
    rows_canon = jnp.concatenate([
        ci.T.reshape(-1),
        inface_idx.astype(jnp.int32),
        jnp.zeros((NCAND - K * C - F,), jnp.int32),
    ])
    iot = jnp.arange(NCAND, dtype=jnp.int32)
    vsorted, order = lax.sort((rows_canon, iot), num_keys=1)
    sortpos = jnp.zeros((NCAND,), jnp.int32).at[order].set(iot)
    bnds = jnp.concatenate([
        jnp.arange(NBF + 1, dtype=jnp.int32) * VB,
        jnp.array([N], jnp.int32)])
    blockstart = jnp.concatenate([
        jnp.searchsorted(vsorted, bnds).astype(jnp.int32),
        jnp.zeros((48 - (NBF + 2),), jnp.int32)])

    mesh = plsc.VectorSubcoreMesh(core_axis_name="c", subcore_axis_name="s")
    run = functools.partial(
        pl.kernel,
        out_type=jax.ShapeDtypeStruct((B * OUTW,), jnp.float32),
        mesh=mesh,
        compiler_params=pltpu.CompilerParams(needs_layout_passes=False,
                                             use_tc_tiling_on_sc=True),
        scratch_types=[
            pltpu.VMEM((NCAND + 32,), jnp.int32),
            pltpu.VMEM((NCAND,), jnp.int32),
            pltpu.VMEM((48,), jnp.int32),
            pltpu.VMEM((8, VB), jnp.float32),
            pltpu.VMEM((8, VB), jnp.float32),
            pltpu.VMEM((8, TAILW), jnp.float32),
            pltpu.VMEM((NCAND * EXTW,), jnp.float32),
            pltpu.VMEM((NCAND * EXTW,), jnp.float32),
            pltpu.VMEM((80,), jnp.int32),
            pltpu.VMEM((8 * OUTW + 32,), jnp.float32),
            pltpu.VMEM_SHARED((8, NCAND * EXTW), jnp.float32),
            pltpu.SemaphoreType.DMA,
            pltpu.SemaphoreType.DMA,
        ],
    )(_sc_body)
    out = run(tab, vsorted.astype(jnp.int32), sortpos, blockstart)
    return out.reshape(B, NLM, 3)
